# Initial kernel scaffold; baseline (speedup 1.0000x reference)
import jax
import jax.numpy as jnp
from jax.experimental import pallas as pl


def kernel(scalar_representation, idx_m, n_atoms, W1, b1, W2, b2):
    raise NotImplementedError("write your pallas kernel here")


# speedup vs baseline: 1.8800x; 1.8800x over previous
import functools

import jax
import jax.numpy as jnp
from jax import lax
from jax.experimental import pallas as pl
from jax.experimental.pallas import tpu as pltpu
from jax.experimental.pallas import tpu_sc as plsc

_NC = 2
_NS = 16
_NW = _NC * _NS
_L = 16


def _mlp_body(x_ref, w1_ref, b1_ref, w2_ref, b2_ref, y_ref):
    x = x_ref[...]
    h = jnp.dot(x, w1_ref[...], preferred_element_type=jnp.float32)
    h = h + b1_ref[...]
    h = h * jax.nn.sigmoid(h)
    y = jnp.dot(h, w2_ref[...], preferred_element_type=jnp.float32)
    y_ref[...] = y + b2_ref[...]


def _mlp(x, w1, b1, w2, b2, blk):
    n, d = x.shape
    h = w1.shape[1]
    grid = n // blk
    return pl.pallas_call(
        _mlp_body,
        grid=(grid,),
        in_specs=[
            pl.BlockSpec((blk, d), lambda i: (i, 0)),
            pl.BlockSpec((d, h), lambda i: (0, 0)),
            pl.BlockSpec((1, h), lambda i: (0, 0)),
            pl.BlockSpec((h, 1), lambda i: (0, 0)),
            pl.BlockSpec((1, 1), lambda i: (0, 0)),
        ],
        out_specs=pl.BlockSpec((blk, 1), lambda i: (i, 0)),
        out_shape=jax.ShapeDtypeStruct((n, 1), jnp.float32),
        compiler_params=pltpu.CompilerParams(
            dimension_semantics=("parallel",)),
    )(x, w1, b1.reshape(1, h), w2, b2.reshape(1, 1))


def _dgather(x, i):
    return lax.gather(
        x, i[:, None],
        lax.GatherDimensionNumbers(
            offset_dims=(), collapsed_slice_dims=(0,), start_index_map=(0,)),
        (1,),
        mode=lax.GatherScatterMode.PROMISE_IN_BOUNDS)


def _make_seg_kernel(n, m):
    chunk = n // _NW
    mesh = plsc.VectorSubcoreMesh(
        core_axis_name="c", subcore_axis_name="s",
        num_cores=_NC, num_subcores=_NS)

    @functools.partial(
        pl.kernel,
        mesh=mesh,
        out_type=jax.ShapeDtypeStruct((_NW, m), jnp.float32),
        scratch_types=[
            pltpu.VMEM((chunk,), jnp.int32),
            pltpu.VMEM((chunk,), jnp.float32),
            pltpu.VMEM((m,), jnp.float32),
        ],
        compiler_params=pltpu.CompilerParams(needs_layout_passes=False),
    )
    def seg(y_hbm, idx_hbm, part_hbm, idx_v, y_v, acc_v):
        wid = lax.axis_index("s") * _NC + lax.axis_index("c")
        base = wid * chunk
        pltpu.sync_copy(idx_hbm.at[pl.ds(base, chunk)], idx_v)
        pltpu.sync_copy(y_hbm.at[pl.ds(base, chunk)], y_v)

        zeros = jnp.zeros((_L,), jnp.float32)

        def zero_body(i, carry):
            acc_v[pl.ds(i * _L, _L)] = zeros
            return carry

        lax.fori_loop(0, m // _L, zero_body, 0, unroll=4)

        lane = lax.iota(jnp.int32, _L)

        def body(i, carry):
            iv = idx_v[pl.ds(i * _L, _L)]
            yv = y_v[pl.ds(i * _L, _L)]
            prev = _dgather(iv, jnp.maximum(lane - 1, 0))
            nxt = _dgather(iv, jnp.minimum(lane + 1, _L - 1))
            m_start = (lane == 0) | (iv != prev)
            m_end = (lane == _L - 1) | (iv != nxt)
            cs = plsc.cumsum(yv)
            startpos = plsc.cummax(jnp.where(m_start, lane, 0))
            subv = _dgather(cs, jnp.maximum(startpos - 1, 0))
            subv = jnp.where(startpos == 0, jnp.zeros_like(subv), subv)
            seg_sums = cs - subv
            plsc.addupdate_scatter(acc_v, [iv], seg_sums, mask=m_end)
            return carry

        lax.fori_loop(0, chunk // _L, body, 0, unroll=4)
        pltpu.sync_copy(acc_v, part_hbm.at[wid])

    return seg


def _combine_body(part_ref, na_ref, out_ref):
    total = jnp.sum(part_ref[...], axis=0, keepdims=True)
    out_ref[...] = total / na_ref[...]


def _combine(partials, n_atoms_f, m):
    return pl.pallas_call(
        _combine_body,
        out_shape=jax.ShapeDtypeStruct((1, m), jnp.float32),
    )(partials, n_atoms_f.reshape(1, m))


def kernel(scalar_representation, idx_m, n_atoms, W1, b1, W2, b2):
    n, d = scalar_representation.shape
    m = n_atoms.shape[0]
    y = _mlp(scalar_representation, W1, b1, W2, b2, blk=4000)
    partials = _make_seg_kernel(n, m)(y.reshape(n), idx_m)
    out = _combine(partials, n_atoms.astype(jnp.float32), m)
    return out.reshape(m)


# speedup vs baseline: 3.9050x; 2.0771x over previous
import functools

import jax
import jax.numpy as jnp
from jax import lax
from jax.experimental import pallas as pl
from jax.experimental.pallas import tpu as pltpu
from jax.experimental.pallas import tpu_sc as plsc

_NC = 2
_NS = 16
_NW = _NC * _NS
_L = 16


def _mlp_body(x_ref, w1_ref, b1_ref, w2_ref, b2_ref, y_ref):
    x = x_ref[...]
    ht = lax.dot_general(
        w1_ref[...], x,
        ((( 0,), (1,)), ((), ())),
        preferred_element_type=jnp.float32)
    ht = ht + b1_ref[...]
    ht = ht * jax.nn.sigmoid(ht)
    y = lax.dot_general(
        w2_ref[...], ht,
        (((0,), (0,)), ((), ())),
        preferred_element_type=jnp.float32)
    y_ref[...] = y.reshape(-1) + b2_ref[0, 0]


def _mlp(x, w1, b1, w2, b2, blk):
    n, d = x.shape
    h = w1.shape[1]
    grid = pl.cdiv(n, blk)
    return pl.pallas_call(
        _mlp_body,
        grid=(grid,),
        in_specs=[
            pl.BlockSpec((blk, d), lambda i: (i, 0)),
            pl.BlockSpec((d, h), lambda i: (0, 0)),
            pl.BlockSpec((h, 1), lambda i: (0, 0)),
            pl.BlockSpec((h, 1), lambda i: (0, 0)),
            pl.BlockSpec((1, 1), lambda i: (0, 0)),
        ],
        out_specs=pl.BlockSpec((blk,), lambda i: (i,)),
        out_shape=jax.ShapeDtypeStruct((n,), jnp.float32),
        compiler_params=pltpu.CompilerParams(
            dimension_semantics=("parallel",)),
    )(x, w1, b1.reshape(h, 1), w2.reshape(h, 1), b2.reshape(1, 1))


def _dgather(x, i):
    return lax.gather(
        x, i[:, None],
        lax.GatherDimensionNumbers(
            offset_dims=(), collapsed_slice_dims=(0,), start_index_map=(0,)),
        (1,),
        mode=lax.GatherScatterMode.PROMISE_IN_BOUNDS)


def _make_seg_kernel(n, m):
    chunk = n // _NW
    mesh = plsc.VectorSubcoreMesh(
        core_axis_name="c", subcore_axis_name="s",
        num_cores=_NC, num_subcores=_NS)

    @functools.partial(
        pl.kernel,
        mesh=mesh,
        out_type=jax.ShapeDtypeStruct((_NW, m), jnp.float32),
        scratch_types=[
            pltpu.VMEM((chunk,), jnp.int32),
            pltpu.VMEM((chunk,), jnp.float32),
            pltpu.VMEM((m,), jnp.float32),
        ],
        compiler_params=pltpu.CompilerParams(needs_layout_passes=False),
    )
    def seg(y_hbm, idx_hbm, part_hbm, idx_v, y_v, acc_v):
        wid = lax.axis_index("s") * _NC + lax.axis_index("c")
        base = wid * chunk
        pltpu.sync_copy(idx_hbm.at[pl.ds(base, chunk)], idx_v)
        pltpu.sync_copy(y_hbm.at[pl.ds(base, chunk)], y_v)

        zeros = jnp.zeros((_L,), jnp.float32)

        def zero_body(i, carry):
            acc_v[pl.ds(i * _L, _L)] = zeros
            return carry

        lax.fori_loop(0, m // _L, zero_body, 0, unroll=4)

        lane = lax.iota(jnp.int32, _L)

        def body(i, carry):
            iv = idx_v[pl.ds(i * _L, _L)]
            yv = y_v[pl.ds(i * _L, _L)]
            prev = _dgather(iv, jnp.maximum(lane - 1, 0))
            nxt = _dgather(iv, jnp.minimum(lane + 1, _L - 1))
            m_start = (lane == 0) | (iv != prev)
            m_end = (lane == _L - 1) | (iv != nxt)
            cs = plsc.cumsum(yv)
            startpos = plsc.cummax(jnp.where(m_start, lane, 0))
            subv = _dgather(cs, jnp.maximum(startpos - 1, 0))
            subv = jnp.where(startpos == 0, jnp.zeros_like(subv), subv)
            seg_sums = cs - subv
            plsc.addupdate_scatter(acc_v, [iv], seg_sums, mask=m_end)
            return carry

        lax.fori_loop(0, chunk // _L, body, 0, unroll=4)
        pltpu.sync_copy(acc_v, part_hbm.at[wid])

    return seg


def _combine_body(part_ref, na_ref, out_ref):
    total = jnp.sum(part_ref[...], axis=0, keepdims=True)
    out_ref[...] = total / na_ref[...]


def _combine(partials, n_atoms_f, m):
    return pl.pallas_call(
        _combine_body,
        out_shape=jax.ShapeDtypeStruct((1, m), jnp.float32),
    )(partials, n_atoms_f.reshape(1, m))


def kernel(scalar_representation, idx_m, n_atoms, W1, b1, W2, b2):
    n, d = scalar_representation.shape
    m = n_atoms.shape[0]
    y = _mlp(scalar_representation, W1, b1, W2, b2, blk=8192)
    partials = _make_seg_kernel(n, m)(y, idx_m)
    out = _combine(partials, n_atoms.astype(jnp.float32), m)
    return out.reshape(m)


# speedup vs baseline: 4.0004x; 1.0244x over previous
import functools

import jax
import jax.numpy as jnp
from jax import lax
from jax.experimental import pallas as pl
from jax.experimental.pallas import tpu as pltpu
from jax.experimental.pallas import tpu_sc as plsc

_NC = 2
_NS = 16
_NW = _NC * _NS
_L = 16


def _mlp_body(x_ref, w1_ref, b1_ref, w2_ref, b2_ref, y_ref):
    x = x_ref[...]
    ht = lax.dot_general(
        w1_ref[...], x,
        ((( 0,), (1,)), ((), ())),
        preferred_element_type=jnp.float32)
    ht = ht + b1_ref[...]
    ht = ht * jax.nn.sigmoid(ht)
    y = lax.dot_general(
        w2_ref[...], ht,
        (((0,), (0,)), ((), ())),
        preferred_element_type=jnp.float32)
    y_ref[...] = y.reshape(-1) + b2_ref[0, 0]


def _mlp(x, w1, b1, w2, b2, blk, base, size):
    n, d = x.shape
    h = w1.shape[1]
    grid = pl.cdiv(size, blk)
    base_blocks = base // blk
    return pl.pallas_call(
        _mlp_body,
        grid=(grid,),
        in_specs=[
            pl.BlockSpec((blk, d), lambda i: (i + base_blocks, 0)),
            pl.BlockSpec((d, h), lambda i: (0, 0)),
            pl.BlockSpec((h, 1), lambda i: (0, 0)),
            pl.BlockSpec((h, 1), lambda i: (0, 0)),
            pl.BlockSpec((1, 1), lambda i: (0, 0)),
        ],
        out_specs=pl.BlockSpec((blk,), lambda i: (i,)),
        out_shape=jax.ShapeDtypeStruct((size,), jnp.float32),
        compiler_params=pltpu.CompilerParams(
            dimension_semantics=("parallel",)),
    )(x, w1, b1.reshape(h, 1), w2.reshape(h, 1), b2.reshape(1, 1))


def _dgather(x, i):
    return lax.gather(
        x, i[:, None],
        lax.GatherDimensionNumbers(
            offset_dims=(), collapsed_slice_dims=(0,), start_index_map=(0,)),
        (1,),
        mode=lax.GatherScatterMode.PROMISE_IN_BOUNDS)


def _make_seg_kernel(base, size, m):
    chunk = size // _NW
    mesh = plsc.VectorSubcoreMesh(
        core_axis_name="c", subcore_axis_name="s",
        num_cores=_NC, num_subcores=_NS)

    @functools.partial(
        pl.kernel,
        mesh=mesh,
        out_type=jax.ShapeDtypeStruct((_NW, m), jnp.float32),
        scratch_types=[
            pltpu.VMEM((chunk,), jnp.int32),
            pltpu.VMEM((chunk,), jnp.float32),
            pltpu.VMEM((m,), jnp.float32),
        ],
        compiler_params=pltpu.CompilerParams(needs_layout_passes=False),
    )
    def seg(y_hbm, idx_hbm, part_hbm, idx_v, y_v, acc_v):
        wid = lax.axis_index("s") * _NC + lax.axis_index("c")
        off = wid * chunk
        pltpu.sync_copy(idx_hbm.at[pl.ds(base + off, chunk)], idx_v)
        pltpu.sync_copy(y_hbm.at[pl.ds(off, chunk)], y_v)

        zeros = jnp.zeros((_L,), jnp.float32)

        def zero_body(i, carry):
            acc_v[pl.ds(i * _L, _L)] = zeros
            return carry

        lax.fori_loop(0, m // _L, zero_body, 0, unroll=4)

        lane = lax.iota(jnp.int32, _L)

        def body(i, carry):
            iv = idx_v[pl.ds(i * _L, _L)]
            yv = y_v[pl.ds(i * _L, _L)]
            prev = _dgather(iv, jnp.maximum(lane - 1, 0))
            nxt = _dgather(iv, jnp.minimum(lane + 1, _L - 1))
            m_start = (lane == 0) | (iv != prev)
            m_end = (lane == _L - 1) | (iv != nxt)
            cs = plsc.cumsum(yv)
            startpos = plsc.cummax(jnp.where(m_start, lane, 0))
            subv = _dgather(cs, jnp.maximum(startpos - 1, 0))
            subv = jnp.where(startpos == 0, jnp.zeros_like(subv), subv)
            seg_sums = cs - subv
            plsc.addupdate_scatter(acc_v, [iv], seg_sums, mask=m_end)
            return carry

        lax.fori_loop(0, chunk // _L, body, 0, unroll=4)
        pltpu.sync_copy(acc_v, part_hbm.at[wid])

    return seg


def _combine_body(*refs):
    na_ref, out_ref = refs[-2], refs[-1]
    total = jnp.sum(refs[0][...], axis=0, keepdims=True)
    for r in refs[1:-2]:
        total = total + jnp.sum(r[...], axis=0, keepdims=True)
    out_ref[...] = total / na_ref[...]


def _combine(partials, n_atoms_f, m):
    return pl.pallas_call(
        _combine_body,
        out_shape=jax.ShapeDtypeStruct((1, m), jnp.float32),
    )(*partials, n_atoms_f.reshape(1, m))


def kernel(scalar_representation, idx_m, n_atoms, W1, b1, W2, b2):
    n, d = scalar_representation.shape
    m = n_atoms.shape[0]
    blk = 8192
    parts = [(0, 163840), (163840, n - 163840)]
    partials = []
    for base, size in parts:
        y_p = _mlp(scalar_representation, W1, b1, W2, b2, blk, base, size)
        partials.append(_make_seg_kernel(base, size, m)(y_p, idx_m))
    out = _combine(partials, n_atoms.astype(jnp.float32), m)
    return out.reshape(m)
